# Initial kernel scaffold; baseline (speedup 1.0000x reference)
import jax
import jax.numpy as jnp
from jax.experimental import pallas as pl


def kernel(x, edge_index, W1, b1, W2, b2):
    raise NotImplementedError("write your pallas kernel here")


# speedup vs baseline: 12.6794x; 12.6794x over previous
import functools

import jax
import jax.numpy as jnp
from jax import lax
from jax.experimental import pallas as pl
from jax.experimental.pallas import tpu as pltpu
from jax.experimental.pallas import tpu_sc as plsc

N = 10000
E = 320000
D_IN = 128
D_HID = 128
D_OUT = 2

NC, NS = 2, 16
NW = NC * NS
CH = 128
NCHUNK = 80
EPT = NCHUNK * CH
E_PAD = NW * EPT
NPAD = 10240
RPT = NPAD // NS
DC = 8
D2 = 16
BLK = 256


def _sc_mesh():
    return plsc.VectorSubcoreMesh(core_axis_name="c", subcore_axis_name="s")


@functools.partial(
    pl.kernel,
    out_type=jax.ShapeDtypeStruct((NC, NPAD, DC), jnp.float32),
    mesh=_sc_mesh(),
    scratch_types=[
        pltpu.VMEM((NCHUNK, CH), jnp.int32),
        pltpu.VMEM((CH, DC), jnp.float32),
        pltpu.VMEM_SHARED((NPAD, DC), jnp.float32),
    ],
    compiler_params=pltpu.CompilerParams(use_tc_tiling_on_sc=False),
)
def _deg_kernel(dst_hbm, ones_hbm, zeros_hbm, out_hbm, dst_v, ones_v, cnt_sh):
    c = lax.axis_index("c")
    s = lax.axis_index("s")
    wid = c * NS + s
    pltpu.sync_copy(zeros_hbm, cnt_sh.at[pl.ds(s * RPT, RPT)])
    pltpu.sync_copy(ones_hbm, ones_v)
    pltpu.sync_copy(dst_hbm.at[wid], dst_v)
    plsc.subcore_barrier()

    def body(j, carry):
        pltpu.sync_copy(ones_v, cnt_sh.at[dst_v.at[j]], add=True)
        return carry

    lax.fori_loop(0, NCHUNK, body, 0)
    plsc.subcore_barrier()
    pltpu.sync_copy(cnt_sh.at[pl.ds(s * RPT, RPT)],
                    out_hbm.at[c, pl.ds(s * RPT, RPT)])


def _make_agg_kernel(d):
    @functools.partial(
        pl.kernel,
        out_type=jax.ShapeDtypeStruct((NC, NPAD, d), jnp.float32),
        mesh=_sc_mesh(),
        scratch_types=[
            pltpu.VMEM((NCHUNK, CH), jnp.int32),
            pltpu.VMEM((NCHUNK, CH), jnp.int32),
            pltpu.VMEM((CH, d), jnp.float32),
            pltpu.SemaphoreType.DMA,
            pltpu.VMEM_SHARED((NPAD, d), jnp.float32),
        ],
        compiler_params=pltpu.CompilerParams(use_tc_tiling_on_sc=False),
    )
    def agg(y_hbm, src_hbm, dst_hbm, zeros_hbm, out_hbm,
            src_v, dst_v, rows_v, sem, acc_sh):
        c = lax.axis_index("c")
        s = lax.axis_index("s")
        wid = c * NS + s
        pltpu.sync_copy(zeros_hbm, acc_sh.at[pl.ds(s * RPT, RPT)])
        pltpu.sync_copy(src_hbm.at[wid], src_v)
        pltpu.sync_copy(dst_hbm.at[wid], dst_v)
        plsc.subcore_barrier()

        def body(j, carry):
            pltpu.async_copy(y_hbm.at[src_v.at[j]], rows_v, sem).wait()
            pltpu.sync_copy(rows_v, acc_sh.at[dst_v.at[j]], add=True)
            return carry

        lax.fori_loop(0, NCHUNK, body, 0)
        plsc.subcore_barrier()
        pltpu.sync_copy(acc_sh.at[pl.ds(s * RPT, RPT)],
                        out_hbm.at[c, pl.ds(s * RPT, RPT)])

    return agg


_agg128 = _make_agg_kernel(D_HID)
_agg16 = _make_agg_kernel(D2)


def _tc_pre(x_pad, W1, c0, c1):
    def body(x_ref, w_ref, c0_ref, c1_ref, y_ref, dinv_ref):
        cnt = c0_ref[...] + c1_ref[...] + 1.0
        dinv = lax.rsqrt(cnt)
        xw = jnp.dot(x_ref[...], w_ref[...], preferred_element_type=jnp.float32)
        y_ref[...] = xw * dinv[:, 0:1]
        dinv_ref[...] = dinv

    return pl.pallas_call(
        body,
        grid=(NPAD // BLK,),
        in_specs=[
            pl.BlockSpec((BLK, D_IN), lambda i: (i, 0)),
            pl.BlockSpec((D_IN, D_HID), lambda i: (0, 0)),
            pl.BlockSpec((BLK, DC), lambda i: (i, 0)),
            pl.BlockSpec((BLK, DC), lambda i: (i, 0)),
        ],
        out_specs=[
            pl.BlockSpec((BLK, D_HID), lambda i: (i, 0)),
            pl.BlockSpec((BLK, DC), lambda i: (i, 0)),
        ],
        out_shape=[
            jax.ShapeDtypeStruct((NPAD, D_HID), jnp.float32),
            jax.ShapeDtypeStruct((NPAD, DC), jnp.float32),
        ],
    )(x_pad, W1, c0, c1)


def _tc_mid(p0, p1, y, dinv, W2p, b1r):
    def body(p0_ref, p1_ref, y_ref, dinv_ref, w_ref, b_ref, z_ref):
        dv = dinv_ref[...][:, 0:1]
        h = dv * (p0_ref[...] + p1_ref[...] + y_ref[...]) + b_ref[...]
        h = jnp.maximum(h, 0.0)
        z_ref[...] = jnp.dot(h, w_ref[...],
                             preferred_element_type=jnp.float32) * dv

    return pl.pallas_call(
        body,
        grid=(NPAD // BLK,),
        in_specs=[
            pl.BlockSpec((BLK, D_HID), lambda i: (i, 0)),
            pl.BlockSpec((BLK, D_HID), lambda i: (i, 0)),
            pl.BlockSpec((BLK, D_HID), lambda i: (i, 0)),
            pl.BlockSpec((BLK, DC), lambda i: (i, 0)),
            pl.BlockSpec((D_HID, D2), lambda i: (0, 0)),
            pl.BlockSpec((1, D_HID), lambda i: (0, 0)),
        ],
        out_specs=pl.BlockSpec((BLK, D2), lambda i: (i, 0)),
        out_shape=jax.ShapeDtypeStruct((NPAD, D2), jnp.float32),
    )(p0, p1, y, dinv, W2p, b1r)


def _tc_post(q0, q1, z, dinv, b2r):
    def body(q0_ref, q1_ref, z_ref, dinv_ref, b_ref, o_ref):
        dv = dinv_ref[...][:, 0:1]
        o_ref[...] = dv * (q0_ref[...] + q1_ref[...] + z_ref[...]) + b_ref[...]

    return pl.pallas_call(
        body,
        grid=(NPAD // BLK,),
        in_specs=[
            pl.BlockSpec((BLK, D2), lambda i: (i, 0)),
            pl.BlockSpec((BLK, D2), lambda i: (i, 0)),
            pl.BlockSpec((BLK, D2), lambda i: (i, 0)),
            pl.BlockSpec((BLK, DC), lambda i: (i, 0)),
            pl.BlockSpec((1, D2), lambda i: (0, 0)),
        ],
        out_specs=pl.BlockSpec((BLK, D2), lambda i: (i, 0)),
        out_shape=jax.ShapeDtypeStruct((NPAD, D2), jnp.float32),
    )(q0, q1, z, dinv, b2r)


def kernel(x, edge_index, W1, b1, W2, b2):
    pad = E_PAD - E
    src3 = jnp.concatenate(
        [edge_index[0], jnp.zeros((pad,), jnp.int32)]).reshape(NW, NCHUNK, CH)
    dst3 = jnp.concatenate(
        [edge_index[1], jnp.full((pad,), N, jnp.int32)]).reshape(NW, NCHUNK, CH)

    ones_c = jnp.ones((CH, DC), jnp.float32)
    zeros_dc = jnp.zeros((RPT, DC), jnp.float32)
    zeros_d1 = jnp.zeros((RPT, D_HID), jnp.float32)
    zeros_d2 = jnp.zeros((RPT, D2), jnp.float32)
    x_pad = jnp.pad(x, ((0, NPAD - N), (0, 0)))
    W2p = jnp.pad(W2, ((0, 0), (0, D2 - D_OUT)))
    b1r = b1.reshape(1, D_HID)
    b2r = jnp.pad(b2, (0, D2 - D_OUT)).reshape(1, D2)

    counts = _deg_kernel(dst3, ones_c, zeros_dc)
    y, dinv = _tc_pre(x_pad, W1, counts[0], counts[1])
    p = _agg128(y, src3, dst3, zeros_d1)
    z = _tc_mid(p[0], p[1], y, dinv, W2p, b1r)
    q = _agg16(z, src3, dst3, zeros_d2)
    out16 = _tc_post(q[0], q[1], z, dinv, b2r)
    return out16[:N, :D_OUT]
